# Initial kernel scaffold; baseline (speedup 1.0000x reference)
import jax
import jax.numpy as jnp
from jax.experimental import pallas as pl


def kernel(feats, mask, seg_quality, W1, b1, W2, b2, Q1, qb1, Q2, qb2):
    raise NotImplementedError("write your pallas kernel here")


# speedup vs baseline: 1.0204x; 1.0204x over previous
import functools

import jax
import jax.numpy as jnp
from jax import lax
from jax.experimental import pallas as pl
from jax.experimental.pallas import tpu as pltpu
from jax.experimental.pallas import tpu_sc as plsc

B, N, D = 64, 2048, 1024
H, QD, QH = 128, 4, 32
K = 64
NEG = -1e9
VALID_THRESH = -1e8


def _score_pool_body(feats_ref, maskf_ref, segq_ref, W1_ref, b1_ref, W2_ref,
                     b2_ref, Q1_ref, qb1_ref, Q2_ref, qb2_ref,
                     logits_ref, weights_ref, attnh_ref):
    f = feats_ref[0]
    h = jnp.tanh(
        jnp.dot(f, W1_ref[...], preferred_element_type=jnp.float32)
        + b1_ref[...])
    ev = lax.dot_general(W2_ref[...], h, (((0,), (1,)), ((), ())),
                         preferred_element_type=jnp.float32)
    q = segq_ref[0]
    qh = jnp.maximum(
        jnp.dot(q, Q1_ref[...], preferred_element_type=jnp.float32)
        + qb1_ref[...], 0.0)
    ql = lax.dot_general(Q2_ref[...], qh, (((0,), (1,)), ((), ())),
                         preferred_element_type=jnp.float32)
    logit = ev + ql + (b2_ref[0, 0] + qb2_ref[0, 0])
    m = maskf_ref[0]
    masked = jnp.where(m > 0.0, logit, NEG)
    logits_ref[0] = masked
    mx = jnp.max(masked)
    e = jnp.exp(masked - mx)
    s = jnp.sum(e)
    w = (e / s) * m
    t = jnp.sum(w)
    w = w / jnp.maximum(t, 1e-8)
    weights_ref[0] = w
    attn = jnp.dot(w, f, preferred_element_type=jnp.float32)
    attnh_ref[0] = 0.5 * attn


def _score_pool(feats, maskf, segq, W1, b1, W2, b2, Q1, qb1, Q2, qb2):
    full = lambda shape: pl.BlockSpec(shape, lambda b: (0,) * len(shape))
    return pl.pallas_call(
        _score_pool_body,
        grid=(B,),
        in_specs=[
            pl.BlockSpec((1, N, D), lambda b: (b, 0, 0)),
            pl.BlockSpec((1, 1, N), lambda b: (b, 0, 0)),
            pl.BlockSpec((1, N, QD), lambda b: (b, 0, 0)),
            full((D, H)), full((1, H)), full((H, 1)), full((1, 1)),
            full((QD, QH)), full((1, QH)), full((QH, 1)), full((1, 1)),
        ],
        out_specs=[
            pl.BlockSpec((1, 1, N), lambda b: (b, 0, 0)),
            pl.BlockSpec((1, 1, N), lambda b: (b, 0, 0)),
            pl.BlockSpec((1, 1, D), lambda b: (b, 0, 0)),
        ],
        out_shape=[
            jax.ShapeDtypeStruct((B, 1, N), jnp.float32),
            jax.ShapeDtypeStruct((B, 1, N), jnp.float32),
            jax.ShapeDtypeStruct((B, 1, D), jnp.float32),
        ],
        compiler_params=pltpu.CompilerParams(
            dimension_semantics=("arbitrary",)),
    )(feats, maskf, segq, W1, b1, W2, b2, Q1, qb1, Q2, qb2)


def _topk_body(logits_ref, idx_ref, validf_ref, fidx_ref, wk_ref):
    cur0 = logits_ref[...]
    iota_n = lax.broadcasted_iota(jnp.int32, (B, N), 1)
    kcol = lax.broadcasted_iota(jnp.int32, (B, K), 1)

    def step(k, carry):
        cur, idxacc, valacc = carry
        mx = jnp.max(cur, axis=1, keepdims=True)
        cand = jnp.where(cur == mx, iota_n, N)
        idx = jnp.min(cand, axis=1, keepdims=True)
        sel = kcol == k
        idxacc = jnp.where(sel, idx, idxacc)
        valacc = jnp.where(sel, mx, valacc)
        cur = jnp.where(iota_n == idx, -jnp.inf, cur)
        return cur, idxacc, valacc

    idxacc0 = jnp.zeros((B, K), jnp.int32)
    valacc0 = jnp.full((B, K), NEG, jnp.float32)
    _, idxacc, valacc = lax.fori_loop(0, K, step, (cur0, idxacc0, valacc0))

    validf = (valacc > VALID_THRESH).astype(jnp.float32)
    cnt = jnp.sum(validf, axis=1, keepdims=True)
    scale = 0.5 / jnp.maximum(cnt, 1.0)
    rowoff = lax.broadcasted_iota(jnp.int32, (B, K), 0) * N
    idx_ref[...] = idxacc
    validf_ref[...] = validf
    fidx_ref[...] = idxacc + rowoff
    wk_ref[...] = jnp.broadcast_to((validf * scale)[:, :, None], (B, K, 16))


def _topk(logits):
    return pl.pallas_call(
        _topk_body,
        out_shape=[
            jax.ShapeDtypeStruct((B, K), jnp.int32),
            jax.ShapeDtypeStruct((B, K), jnp.float32),
            jax.ShapeDtypeStruct((B, K), jnp.int32),
            jax.ShapeDtypeStruct((B, K, 16), jnp.float32),
        ],
    )(logits)


def _sc_gather_body(feats_hbm, fidx_hbm, wk_hbm, attnh_hbm, out_hbm,
                    idx_v, rows_v, wk_v, acc_v, sem):
    wid = lax.axis_index("s") * 2 + lax.axis_index("c")
    for t in range(2):
        b = wid * 2 + t
        pltpu.sync_copy(fidx_hbm.at[b], idx_v)
        cp = pltpu.async_copy(feats_hbm.at[idx_v], rows_v, sem)
        pltpu.sync_copy(wk_hbm.at[b], wk_v)
        pltpu.sync_copy(attnh_hbm.at[b], acc_v)
        cp.wait()

        def krow(k, _):
            wkv = wk_v[k, :]
            for c in range(D // 16):
                sl = pl.ds(c * 16, 16)
                acc_v[sl] = acc_v[sl] + rows_v[k, sl] * wkv
            return 0

        lax.fori_loop(0, K, krow, 0)
        pltpu.sync_copy(acc_v, out_hbm.at[b])


@functools.cache
def _sc_gather_kernel():
    return pl.kernel(
        _sc_gather_body,
        out_type=jax.ShapeDtypeStruct((B, D), jnp.float32),
        mesh=plsc.VectorSubcoreMesh(
            core_axis_name="c", subcore_axis_name="s",
            num_cores=2, num_subcores=16),
        scratch_types=[
            pltpu.VMEM((K,), jnp.int32),
            pltpu.VMEM((K, D), jnp.float32),
            pltpu.VMEM((K, 16), jnp.float32),
            pltpu.VMEM((D,), jnp.float32),
            pltpu.SemaphoreType.DMA,
        ],
    )


def kernel(feats, mask, seg_quality, W1, b1, W2, b2, Q1, qb1, Q2, qb2):
    maskf = mask.astype(jnp.float32).reshape(B, 1, N)
    logits, weights, attnh = _score_pool(
        feats, maskf, seg_quality, W1,
        b1.reshape(1, H), W2, b2.reshape(1, 1),
        Q1, qb1.reshape(1, QH), Q2, qb2.reshape(1, 1))
    logits = logits.reshape(B, N)
    weights = weights.reshape(B, N)
    attnh = attnh.reshape(B, D)
    topk_idx, validf, fidx, wk = _topk(logits)
    feats2d = feats.reshape(B * N, D)
    pooled = _sc_gather_kernel()(feats2d, fidx, wk, attnh)
    return (pooled, weights, topk_idx, validf.astype(bool))


# speedup vs baseline: 1.1556x; 1.1325x over previous
import functools

import jax
import jax.numpy as jnp
from jax import lax
from jax.experimental import pallas as pl
from jax.experimental.pallas import tpu as pltpu
from jax.experimental.pallas import tpu_sc as plsc

B, N, D = 64, 2048, 1024
H, QD, QH = 128, 4, 32
K = 64
NEG = -1e9
VALID_THRESH = -1e8


def _score_pool_body(feats_ref, maskf_ref, segq_ref, W1_ref, b1_ref, W2_ref,
                     b2_ref, Q1_ref, qb1_ref, Q2_ref, qb2_ref,
                     weights_ref, attnh_ref, idx_ref, validf_ref, fidx_ref,
                     wk_ref, logits_sc):
    b = pl.program_id(0)
    f = feats_ref[0]
    h = jnp.tanh(
        jnp.dot(f, W1_ref[...], preferred_element_type=jnp.float32)
        + b1_ref[...])
    ev = lax.dot_general(W2_ref[...], h, (((0,), (1,)), ((), ())),
                         preferred_element_type=jnp.float32)
    q = segq_ref[0]
    qh = jnp.maximum(
        jnp.dot(q, Q1_ref[...], preferred_element_type=jnp.float32)
        + qb1_ref[...], 0.0)
    ql = lax.dot_general(Q2_ref[...], qh, (((0,), (1,)), ((), ())),
                         preferred_element_type=jnp.float32)
    logit = ev + ql + (b2_ref[0, 0] + qb2_ref[0, 0])
    m = maskf_ref[0]
    masked = jnp.where(m > 0.0, logit, NEG)
    logits_sc[pl.ds(b, 1)] = masked
    mx = jnp.max(masked)
    e = jnp.exp(masked - mx)
    s = jnp.sum(e)
    w = (e / s) * m
    t = jnp.sum(w)
    w = w / jnp.maximum(t, 1e-8)
    weights_ref[0] = w
    attn = jnp.dot(w, f, preferred_element_type=jnp.float32)
    attnh_ref[0] = 0.5 * attn

    @pl.when(b == B - 1)
    def _topk_tail():
        cur0 = logits_sc[...]
        iota_n = lax.broadcasted_iota(jnp.int32, (B, N), 1)
        kcol = lax.broadcasted_iota(jnp.int32, (B, K), 1)

        def step(k, carry):
            cur, idxacc, valacc = carry
            mxk = jnp.max(cur, axis=1, keepdims=True)
            cand = jnp.where(cur == mxk, iota_n, N)
            idx = jnp.min(cand, axis=1, keepdims=True)
            sel = kcol == k
            idxacc = jnp.where(sel, idx, idxacc)
            valacc = jnp.where(sel, mxk, valacc)
            cur = jnp.where(iota_n == idx, -jnp.inf, cur)
            return cur, idxacc, valacc

        idxacc0 = jnp.zeros((B, K), jnp.int32)
        valacc0 = jnp.full((B, K), NEG, jnp.float32)
        _, idxacc, valacc = lax.fori_loop(
            0, K, step, (cur0, idxacc0, valacc0))

        validf = (valacc > VALID_THRESH).astype(jnp.float32)
        cnt = jnp.sum(validf, axis=1, keepdims=True)
        scale = 0.5 / jnp.maximum(cnt, 1.0)
        rowoff = lax.broadcasted_iota(jnp.int32, (B, K), 0) * N
        idx_ref[...] = idxacc
        validf_ref[...] = validf
        fidx_ref[...] = idxacc + rowoff
        wk_ref[...] = jnp.broadcast_to(
            (validf * scale)[:, :, None], (B, K, 16))


def _score_pool(feats, maskf, segq, W1, b1, W2, b2, Q1, qb1, Q2, qb2):
    full = lambda shape: pl.BlockSpec(shape, lambda b: (0,) * len(shape))
    return pl.pallas_call(
        _score_pool_body,
        grid=(B,),
        in_specs=[
            pl.BlockSpec((1, N, D), lambda b: (b, 0, 0)),
            pl.BlockSpec((1, 1, N), lambda b: (b, 0, 0)),
            pl.BlockSpec((1, N, QD), lambda b: (b, 0, 0)),
            full((D, H)), full((1, H)), full((H, 1)), full((1, 1)),
            full((QD, QH)), full((1, QH)), full((QH, 1)), full((1, 1)),
        ],
        out_specs=[
            pl.BlockSpec((1, 1, N), lambda b: (b, 0, 0)),
            pl.BlockSpec((1, 1, D), lambda b: (b, 0, 0)),
            pl.BlockSpec((B, K), lambda b: (0, 0)),
            pl.BlockSpec((B, K), lambda b: (0, 0)),
            pl.BlockSpec((B, K), lambda b: (0, 0)),
            pl.BlockSpec((B, K, 16), lambda b: (0, 0, 0)),
        ],
        out_shape=[
            jax.ShapeDtypeStruct((B, 1, N), jnp.float32),
            jax.ShapeDtypeStruct((B, 1, D), jnp.float32),
            jax.ShapeDtypeStruct((B, K), jnp.int32),
            jax.ShapeDtypeStruct((B, K), jnp.float32),
            jax.ShapeDtypeStruct((B, K), jnp.int32),
            jax.ShapeDtypeStruct((B, K, 16), jnp.float32),
        ],
        scratch_shapes=[pltpu.VMEM((B, N), jnp.float32)],
        compiler_params=pltpu.CompilerParams(
            dimension_semantics=("arbitrary",)),
    )(feats, maskf, segq, W1, b1, W2, b2, Q1, qb1, Q2, qb2)


def _sc_gather_body(feats_hbm, fidx_hbm, wk_hbm, attnh_hbm, out_hbm,
                    idx_v, rows_v, wk_v, acc_v, sem):
    wid = lax.axis_index("s") * 2 + lax.axis_index("c")
    GW = 16
    for t in range(2):
        b = wid * 2 + t
        pltpu.sync_copy(fidx_hbm.at[b], idx_v)
        cp = pltpu.async_copy(feats_hbm.at[idx_v], rows_v, sem)
        pltpu.sync_copy(wk_hbm.at[b], wk_v)
        pltpu.sync_copy(attnh_hbm.at[b], acc_v)
        cp.wait()

        for g in range(D // (16 * GW)):
            base = g * 16 * GW

            def kbody(k, accs):
                wkv = wk_v[k, :]
                return tuple(
                    accs[i] + rows_v[k, pl.ds(base + i * 16, 16)] * wkv
                    for i in range(GW))

            acc0 = tuple(
                acc_v[pl.ds(base + i * 16, 16)] for i in range(GW))
            accs = lax.fori_loop(0, K, kbody, acc0)
            for i in range(GW):
                acc_v[pl.ds(base + i * 16, 16)] = accs[i]

        pltpu.sync_copy(acc_v, out_hbm.at[b])


@functools.cache
def _sc_gather_kernel():
    return pl.kernel(
        _sc_gather_body,
        out_type=jax.ShapeDtypeStruct((B, D), jnp.float32),
        mesh=plsc.VectorSubcoreMesh(
            core_axis_name="c", subcore_axis_name="s",
            num_cores=2, num_subcores=16),
        scratch_types=[
            pltpu.VMEM((K,), jnp.int32),
            pltpu.VMEM((K, D), jnp.float32),
            pltpu.VMEM((K, 16), jnp.float32),
            pltpu.VMEM((D,), jnp.float32),
            pltpu.SemaphoreType.DMA,
        ],
    )


def kernel(feats, mask, seg_quality, W1, b1, W2, b2, Q1, qb1, Q2, qb2):
    maskf = mask.astype(jnp.float32).reshape(B, 1, N)
    weights, attnh, topk_idx, validf, fidx, wk = _score_pool(
        feats, maskf, seg_quality, W1,
        b1.reshape(1, H), W2, b2.reshape(1, 1),
        Q1, qb1.reshape(1, QH), Q2, qb2.reshape(1, 1))
    weights = weights.reshape(B, N)
    attnh = attnh.reshape(B, D)
    feats2d = feats.reshape(B * N, D)
    pooled = _sc_gather_kernel()(feats2d, fidx, wk, attnh)
    return (pooled, weights, topk_idx, validf.astype(bool))
